# baseline (speedup 1.0000x reference)
import functools

import jax
import jax.numpy as jnp
from jax import lax
from jax.experimental import pallas as pl
from jax.experimental.pallas import tpu as pltpu
from jax.experimental.pallas import tpu_sc as plsc

SCALE = 32
NG = 2 * SCALE
NG3 = NG * NG * NG

_TAP_OFFS = tuple(dx * NG * NG + dy * NG + dz
                  for dx in (0, 1) for dy in (0, 1) for dz in (0, 1))

L = 16
CHUNK = 2048
NBUF = 2


def _make_sampler(B, N):
    NW = 32
    assert B == NW and N % (CHUNK * NBUF) == 0
    nch = N // CHUNK
    mesh = plsc.VectorSubcoreMesh(core_axis_name="c", subcore_axis_name="s")

    _GDN = lax.GatherDimensionNumbers(offset_dims=(), collapsed_slice_dims=(0,),
                                      start_index_map=(0,))

    def _vgather(a, idx):
        return lax.gather(a, idx[:, None], dimension_numbers=_GDN,
                          slice_sizes=(1,),
                          mode=lax.GatherScatterMode.PROMISE_IN_BOUNDS)

    def _deint(A, Bv, Cv, coord):
        pos = 3 * lax.iota(jnp.int32, L) + coord
        sub = pos & (L - 1)
        reg = pos >> 4
        ga = _vgather(A, sub)
        gb = _vgather(Bv, sub)
        gc = _vgather(Cv, sub)
        return jnp.where(reg == 0, ga, jnp.where(reg == 1, gb, gc))

    @functools.partial(
        pl.kernel,
        mesh=mesh,
        out_type=jax.ShapeDtypeStruct((B * N,), jnp.float32),
        scratch_types=(
            [pltpu.VMEM((3 * CHUNK,), jnp.float32) for _ in range(NBUF)]
            + [pltpu.VMEM((3, CHUNK), jnp.float32) for _ in range(NBUF)]
            + [pltpu.VMEM((8 * CHUNK,), jnp.int32) for _ in range(NBUF)]
            + [pltpu.VMEM((8 * CHUNK,), jnp.float32) for _ in range(NBUF)]
            + [pltpu.VMEM((CHUNK,), jnp.float32) for _ in range(NBUF)]
            + [pltpu.SemaphoreType.DMA for _ in range(3 * NBUF)]
        ),
    )
    def sampler(grid_hbm, pts_hbm, out_hbm, *bufs):
        xyz = bufs[0:2]
        tbuf = bufs[2:4]
        idx = bufs[4:6]
        val = bufs[6:8]
        outv = bufs[8:10]
        sem_xyz = bufs[10:12]
        sem_g = bufs[12:14]
        sem_o = bufs[14:16]

        wid = lax.axis_index("s") * 2 + lax.axis_index("c")
        pbase = wid * N
        gbase = wid * NG3

        def start_xyz(ci, b):
            pltpu.async_copy(pts_hbm.at[pl.ds(3 * (pbase + ci * CHUNK), 3 * CHUNK)],
                             xyz[b], sem_xyz[b])

        def pass1(b):
            def grp(i, _):
                o = 3 * i * L
                A = xyz[b][pl.ds(o, L)]
                Bv = xyz[b][pl.ds(o + L, L)]
                Cv = xyz[b][pl.ds(o + 2 * L, L)]
                xs = _deint(A, Bv, Cv, 0) + float(SCALE)
                ys = _deint(A, Bv, Cv, 1) + float(SCALE)
                zs = _deint(A, Bv, Cv, 2) + float(SCALE)
                o = i * L
                fi = xs.astype(jnp.int32)
                fj = ys.astype(jnp.int32)
                fk = zs.astype(jnp.int32)
                tbuf[b][0, pl.ds(o, L)] = xs - fi.astype(jnp.float32)
                tbuf[b][1, pl.ds(o, L)] = ys - fj.astype(jnp.float32)
                tbuf[b][2, pl.ds(o, L)] = zs - fk.astype(jnp.float32)
                ii = jnp.minimum(fi, NG - 2)
                jj = jnp.minimum(fj, NG - 2)
                kk = jnp.minimum(fk, NG - 2)
                lin0 = (ii << 12) + (jj << 6) + kk + gbase
                for t in range(8):
                    idx[b][pl.ds(t * CHUNK + o, L)] = lin0 + _TAP_OFFS[t]

            lax.fori_loop(0, CHUNK // L, grp, None, unroll=2)

        def pass2(b):
            def grp(i, _):
                o = i * L
                tx = tbuf[b][0, pl.ds(o, L)]
                ty = tbuf[b][1, pl.ds(o, L)]
                tz = tbuf[b][2, pl.ds(o, L)]
                v = [val[b][pl.ds(t * CHUNK + o, L)] for t in range(8)]
                c00 = v[0] + tz * (v[1] - v[0])
                c01 = v[2] + tz * (v[3] - v[2])
                c10 = v[4] + tz * (v[5] - v[4])
                c11 = v[6] + tz * (v[7] - v[6])
                d0 = c00 + ty * (c01 - c00)
                d1 = c10 + ty * (c11 - c10)
                outv[b][pl.ds(o, L)] = d0 + tx * (d1 - d0)

            lax.fori_loop(0, CHUNK // L, grp, None, unroll=2)

        def start_gather(b):
            pltpu.async_copy(grid_hbm.at[idx[b]], val[b], sem_g[b])

        def start_out(ci, b):
            pltpu.async_copy(outv[b], out_hbm.at[pl.ds(pbase + ci * CHUNK, CHUNK)],
                             sem_o[b])

        def wait_xyz(b):
            pltpu.make_async_copy(pts_hbm.at[pl.ds(0, 3 * CHUNK)], xyz[b],
                                  sem_xyz[b]).wait()

        def wait_gather(b):
            pltpu.make_async_copy(grid_hbm.at[idx[b]], val[b], sem_g[b]).wait()

        def wait_out(b):
            pltpu.make_async_copy(outv[b], out_hbm.at[pl.ds(0, CHUNK)],
                                  sem_o[b]).wait()

        start_xyz(0, 0)
        start_xyz(1, 1)
        wait_xyz(0)
        pass1(0)
        start_gather(0)

        def step(ci, cur):
            nxt = 1 - cur
            @pl.when(ci + 1 < nch)
            def _():
                wait_xyz(nxt)
                pass1(nxt)
                start_gather(nxt)

            @pl.when(ci + 2 < nch)
            def _():
                start_xyz(ci + 2, cur)

            @pl.when(ci >= 2)
            def _():
                wait_out(cur)

            wait_gather(cur)
            pass2(cur)
            start_out(ci, cur)

        def two_steps(m, _):
            step(2 * m, 0)
            step(2 * m + 1, 1)
            return None

        lax.fori_loop(0, nch // 2, two_steps, None)
        wait_out(0)
        wait_out(1)

    return sampler


def kernel(grid, ptcloud):
    B, N = ptcloud.shape[0], ptcloud.shape[1]
    out = _make_sampler(B, N)(grid.reshape(-1), ptcloud.reshape(-1))
    return out.reshape(B, N)


# speedup vs baseline: 16.0927x; 16.0927x over previous
import functools

import jax
import jax.numpy as jnp
from jax import lax
from jax.experimental import pallas as pl
from jax.experimental.pallas import tpu as pltpu
from jax.experimental.pallas import tpu_sc as plsc

SCALE = 32
NG = 2 * SCALE
NG3 = NG * NG * NG

_PAIR_OFFS = (0, NG, NG * NG, NG * NG + NG)

L = 16
CHUNK = 2048
NBUF = 2

_PACK_COLS = 1024
_PACK_ROWS = 512


def _pack_body(g_ref, o_ref):
    x = g_ref[...]
    wrapcol = jnp.concatenate([x[1:, :1], x[:1, :1]], axis=0)
    xs = jnp.concatenate([x[:, 1:], wrapcol], axis=1)
    lo = lax.bitcast_convert_type(x.astype(jnp.bfloat16),
                                  jnp.uint16).astype(jnp.uint32)
    hi = lax.bitcast_convert_type(xs.astype(jnp.bfloat16),
                                  jnp.uint16).astype(jnp.uint32)
    o_ref[...] = (hi << 16) | lo


def _pack_pairs(gflat):
    n = gflat.shape[0]
    rows = n // _PACK_COLS
    g2 = gflat.reshape(rows, _PACK_COLS)
    out = pl.pallas_call(
        _pack_body,
        grid=(rows // _PACK_ROWS,),
        in_specs=[pl.BlockSpec((_PACK_ROWS, _PACK_COLS), lambda i: (i, 0))],
        out_specs=pl.BlockSpec((_PACK_ROWS, _PACK_COLS), lambda i: (i, 0)),
        out_shape=jax.ShapeDtypeStruct((rows, _PACK_COLS), jnp.uint32),
    )(g2)
    return out.reshape(-1)


def _make_sampler(B, N):
    NW = 32
    assert B == NW and N % (CHUNK * NBUF) == 0
    nch = N // CHUNK
    mesh = plsc.VectorSubcoreMesh(core_axis_name="c", subcore_axis_name="s")

    @functools.partial(
        pl.kernel,
        mesh=mesh,
        out_type=jax.ShapeDtypeStruct((B * N,), jnp.float32),
        scratch_types=(
            [pltpu.VMEM((3, CHUNK), jnp.float32) for _ in range(NBUF)]
            + [pltpu.VMEM((3, CHUNK), jnp.float32) for _ in range(NBUF)]
            + [pltpu.VMEM((4 * CHUNK,), jnp.int32) for _ in range(NBUF)]
            + [pltpu.VMEM((4 * CHUNK,), jnp.uint32) for _ in range(NBUF)]
            + [pltpu.VMEM((CHUNK,), jnp.float32) for _ in range(NBUF)]
            + [pltpu.SemaphoreType.DMA for _ in range(3 * NBUF)]
        ),
    )
    def sampler(tb_hbm, pts_hbm, out_hbm, *bufs):
        xyz = bufs[0:2]
        tbuf = bufs[2:4]
        idx = bufs[4:6]
        val = bufs[6:8]
        outv = bufs[8:10]
        sem_xyz = bufs[10:12]
        sem_g = bufs[12:14]
        sem_o = bufs[14:16]

        wid = lax.axis_index("s") * 2 + lax.axis_index("c")
        pbase = wid * N
        gbase = wid * NG3

        def start_xyz(ci, b):
            pltpu.async_copy(pts_hbm.at[:, pl.ds(pbase + ci * CHUNK, CHUNK)],
                             xyz[b], sem_xyz[b])

        def pass1(b):
            def grp(i, _):
                o = i * L
                xs = xyz[b][0, pl.ds(o, L)] + float(SCALE)
                ys = xyz[b][1, pl.ds(o, L)] + float(SCALE)
                zs = xyz[b][2, pl.ds(o, L)] + float(SCALE)
                fi = xs.astype(jnp.int32)
                fj = ys.astype(jnp.int32)
                fk = zs.astype(jnp.int32)
                tbuf[b][0, pl.ds(o, L)] = xs - fi.astype(jnp.float32)
                tbuf[b][1, pl.ds(o, L)] = ys - fj.astype(jnp.float32)
                tbuf[b][2, pl.ds(o, L)] = zs - fk.astype(jnp.float32)
                ii = jnp.minimum(fi, NG - 2)
                jj = jnp.minimum(fj, NG - 2)
                kk = jnp.minimum(fk, NG - 2)
                lin0 = (ii << 12) + (jj << 6) + kk + gbase
                for t in range(4):
                    idx[b][pl.ds(t * CHUNK + o, L)] = lin0 + _PAIR_OFFS[t]

            lax.fori_loop(0, CHUNK // L, grp, None, unroll=2)

        def pass2(b):
            def grp(i, _):
                o = i * L
                tx = tbuf[b][0, pl.ds(o, L)]
                ty = tbuf[b][1, pl.ds(o, L)]
                tz = tbuf[b][2, pl.ds(o, L)]
                c = []
                for t in range(4):
                    pv = val[b][pl.ds(t * CHUNK + o, L)]
                    v0 = lax.bitcast_convert_type(pv << 16, jnp.float32)
                    v1 = lax.bitcast_convert_type(
                        pv & jnp.uint32(0xFFFF0000), jnp.float32)
                    c.append(v0 + tz * (v1 - v0))
                d0 = c[0] + ty * (c[1] - c[0])
                d1 = c[2] + ty * (c[3] - c[2])
                outv[b][pl.ds(o, L)] = d0 + tx * (d1 - d0)

            lax.fori_loop(0, CHUNK // L, grp, None, unroll=2)

        def start_gather(b):
            pltpu.async_copy(tb_hbm.at[idx[b]], val[b], sem_g[b])

        def start_out(ci, b):
            pltpu.async_copy(outv[b], out_hbm.at[pl.ds(pbase + ci * CHUNK, CHUNK)],
                             sem_o[b])

        def wait_xyz(b):
            pltpu.make_async_copy(pts_hbm.at[:, pl.ds(0, CHUNK)], xyz[b],
                                  sem_xyz[b]).wait()

        def wait_gather(b):
            pltpu.make_async_copy(tb_hbm.at[idx[b]], val[b], sem_g[b]).wait()

        def wait_out(b):
            pltpu.make_async_copy(outv[b], out_hbm.at[pl.ds(0, CHUNK)],
                                  sem_o[b]).wait()

        start_xyz(0, 0)
        start_xyz(1, 1)
        wait_xyz(0)
        pass1(0)
        start_gather(0)

        def step(ci, cur):
            nxt = 1 - cur

            @pl.when(ci + 1 < nch)
            def _():
                wait_xyz(nxt)
                pass1(nxt)
                start_gather(nxt)

            @pl.when(ci + 2 < nch)
            def _():
                start_xyz(ci + 2, cur)

            @pl.when(ci >= 2)
            def _():
                wait_out(cur)

            wait_gather(cur)
            pass2(cur)
            start_out(ci, cur)

        def two_steps(m, _):
            step(2 * m, 0)
            step(2 * m + 1, 1)
            return None

        lax.fori_loop(0, nch // 2, two_steps, None)
        wait_out(0)
        wait_out(1)

    return sampler


def kernel(grid, ptcloud):
    B, N = ptcloud.shape[0], ptcloud.shape[1]
    tb = _pack_pairs(grid.reshape(-1))
    pts = ptcloud.reshape(B * N, 3).T
    out = _make_sampler(B, N)(tb, pts)
    return out.reshape(B, N)
